# baseline (speedup 1.0000x reference)
import functools

import jax
import jax.numpy as jnp
import numpy as np
from jax import lax
from jax.experimental import pallas as pl
from jax.experimental.pallas import tpu as pltpu
from jax.experimental.pallas import tpu_sc as plsc


def _gather_sparsecore(idx_flat, flat_table):
    n = idx_flat.shape[0]
    info = plsc.get_sparse_core_info()
    nc, ns = info.num_cores, info.num_subcores
    nw = nc * ns
    n_per_w = n // nw

    mesh = plsc.VectorSubcoreMesh(core_axis_name="c", subcore_axis_name="s")

    @functools.partial(
        pl.kernel,
        mesh=mesh,
        out_type=jax.ShapeDtypeStruct((n,), jnp.float32),
        scratch_types=[
            pltpu.VMEM((n_per_w,), jnp.int32),
            pltpu.VMEM((n_per_w,), jnp.float32),
            pltpu.SemaphoreType.DMA,
        ],
    )
    def gather_kernel(idx_hbm, table_hbm, out_hbm, idx_v, w_v, sem):
        wid = lax.axis_index("s") * nc + lax.axis_index("c")
        base = wid * n_per_w
        pltpu.sync_copy(idx_hbm.at[pl.ds(base, n_per_w)], idx_v)
        pltpu.async_copy(table_hbm.at[idx_v], w_v, sem).wait()
        pltpu.sync_copy(w_v, out_hbm.at[pl.ds(base, n_per_w)])

    return gather_kernel(idx_flat, flat_table)


def _interaction_tc(emb2d, sel):
    batch, fd = emb2d.shape
    d = sel.shape[1]
    blk = 1024
    rpb = blk // 128

    def body(emb_ref, sel_ref, out_ref):
        x = emb_ref[...]
        sum_sq = jnp.sum(x * x, axis=1)
        s = jnp.dot(x, sel_ref[...],
                    preferred_element_type=jnp.float32)
        dots = 0.5 * (jnp.sum(s * s, axis=1) - sum_sq)
        out_ref[...] = dots.reshape(rpb, 128)

    return pl.pallas_call(
        body,
        grid=(batch // blk,),
        in_specs=[
            pl.BlockSpec((blk, fd), lambda i: (i, 0)),
            pl.BlockSpec((fd, d), lambda i: (0, 0)),
        ],
        out_specs=pl.BlockSpec((rpb, 128), lambda i: (i, 0)),
        out_shape=jax.ShapeDtypeStruct((batch // 128, 128), jnp.float32),
    )(emb2d, sel)


def _combine_tc(dots128, w_wide, k_sel, bias11):
    rows, wide = w_wide.shape
    rpb = 8

    def body(d_ref, w_ref, k_ref, bias_ref, out_ref):
        lr = jnp.dot(w_ref[...], k_ref[...],
                     preferred_element_type=jnp.float32)
        out_ref[...] = d_ref[...] + lr + bias_ref[0, 0]

    del rpb
    return pl.pallas_call(
        body,
        grid=(1,),
        in_specs=[
            pl.BlockSpec((rows, 128), lambda i: (0, 0)),
            pl.BlockSpec((rows, wide), lambda i: (0, 0)),
            pl.BlockSpec((wide, 128), lambda i: (0, 0)),
            pl.BlockSpec((1, 1), lambda i: (0, 0)),
        ],
        out_specs=pl.BlockSpec((rows, 128), lambda i: (0, 0)),
        out_shape=jax.ShapeDtypeStruct((rows, 128), jnp.float32),
    )(dots128, w_wide, k_sel, bias11)


def kernel(X, feature_emb, lr_table, bias):
    batch, nfields = X.shape
    vocab = lr_table.shape[1]
    d = feature_emb.shape[2]
    fd = nfields * d

    n = batch * nfields
    field_off = (np.arange(n, dtype=np.int32) % nfields) * vocab
    idx_flat = X.reshape(-1) + jnp.asarray(field_off)
    flat_table = jnp.concatenate(
        [lr_table[f] for f in range(nfields)])

    w_flat = _gather_sparsecore(idx_flat, flat_table)

    sel = jnp.asarray(
        np.tile(np.eye(d, dtype=np.float32), (nfields, 1)))
    wide = 128 * nfields
    k_sel = jnp.asarray(
        (np.arange(wide)[:, None] // nfields
         == np.arange(128)[None, :]).astype(np.float32))

    emb2d = feature_emb.reshape(batch, fd)
    dots128 = _interaction_tc(emb2d, sel)
    out128 = _combine_tc(dots128, w_flat.reshape(batch // 128, wide),
                         k_sel, bias.reshape(1, 1))
    return out128.reshape(batch, 1)


# speedup vs baseline: 2.3422x; 2.3422x over previous
import functools

import jax
import jax.numpy as jnp
import numpy as np
from jax import lax
from jax.experimental import pallas as pl
from jax.experimental.pallas import tpu as pltpu
from jax.experimental.pallas import tpu_sc as plsc


def _gather_sparsecore(idx_flat, flat_table):
    n = idx_flat.shape[0]
    info = plsc.get_sparse_core_info()
    nc, ns = info.num_cores, info.num_subcores
    nw = nc * ns
    n_per_w = n // nw

    mesh = plsc.VectorSubcoreMesh(core_axis_name="c", subcore_axis_name="s")

    @functools.partial(
        pl.kernel,
        mesh=mesh,
        out_type=jax.ShapeDtypeStruct((n,), jnp.float32),
        scratch_types=[
            pltpu.VMEM((n_per_w,), jnp.int32),
            pltpu.VMEM((n_per_w,), jnp.float32),
            pltpu.SemaphoreType.DMA,
        ],
    )
    def gather_kernel(idx_hbm, table_hbm, out_hbm, idx_v, w_v, sem):
        wid = lax.axis_index("s") * nc + lax.axis_index("c")
        base = wid * n_per_w
        pltpu.sync_copy(idx_hbm.at[pl.ds(base, n_per_w)], idx_v)
        pltpu.async_copy(table_hbm.at[idx_v], w_v, sem).wait()
        pltpu.sync_copy(w_v, out_hbm.at[pl.ds(base, n_per_w)])

    return gather_kernel(idx_flat, flat_table)


def _interaction_tc(emb2d, sel):
    batch, fd = emb2d.shape
    d = sel.shape[1]
    blk = 1024
    rpb = blk // 128

    def body(emb_ref, sel_ref, out_ref):
        x = emb_ref[...]
        sum_sq = jnp.sum(x * x, axis=1)
        s = jnp.dot(x, sel_ref[...],
                    preferred_element_type=jnp.float32)
        dots = 0.5 * (jnp.sum(s * s, axis=1) - sum_sq)
        out_ref[...] = dots.reshape(rpb, 128)

    return pl.pallas_call(
        body,
        grid=(batch // blk,),
        in_specs=[
            pl.BlockSpec((blk, fd), lambda i: (i, 0)),
            pl.BlockSpec((fd, d), lambda i: (0, 0)),
        ],
        out_specs=pl.BlockSpec((rpb, 128), lambda i: (i, 0)),
        out_shape=jax.ShapeDtypeStruct((batch // 128, 128), jnp.float32),
    )(emb2d, sel)


def _combine_tc(dots128, w_wide, k_sel, bias11):
    rows, wide = w_wide.shape
    rpb = 8

    def body(d_ref, w_ref, k_ref, bias_ref, out_ref):
        lr = jnp.dot(w_ref[...], k_ref[...],
                     preferred_element_type=jnp.float32)
        out_ref[...] = d_ref[...] + lr + bias_ref[0, 0]

    del rpb
    return pl.pallas_call(
        body,
        grid=(1,),
        in_specs=[
            pl.BlockSpec((rows, 128), lambda i: (0, 0)),
            pl.BlockSpec((rows, wide), lambda i: (0, 0)),
            pl.BlockSpec((wide, 128), lambda i: (0, 0)),
            pl.BlockSpec((1, 1), lambda i: (0, 0)),
        ],
        out_specs=pl.BlockSpec((rows, 128), lambda i: (0, 0)),
        out_shape=jax.ShapeDtypeStruct((rows, 128), jnp.float32),
    )(dots128, w_wide, k_sel, bias11)


def kernel(X, feature_emb, lr_table, bias):
    batch, nfields = X.shape
    vocab = lr_table.shape[1]
    d = feature_emb.shape[2]
    fd = nfields * d

    n = batch * nfields
    field_off = (np.arange(n, dtype=np.int32) % nfields) * vocab
    idx_flat = X.reshape(-1) + jnp.asarray(field_off)

    w_flat = _gather_sparsecore(idx_flat, lr_table.reshape(-1))

    sel = jnp.asarray(
        np.tile(np.eye(d, dtype=np.float32), (nfields, 1)))
    wide = 128 * nfields
    k_sel = jnp.asarray(
        (np.arange(wide)[:, None] // nfields
         == np.arange(128)[None, :]).astype(np.float32))

    emb2d = feature_emb.reshape(batch, fd)
    dots128 = _interaction_tc(emb2d, sel)
    out128 = _combine_tc(dots128, w_flat.reshape(batch // 128, wide),
                         k_sel, bias.reshape(1, 1))
    return out128.reshape(batch, 1)


# speedup vs baseline: 2.8815x; 1.2303x over previous
import functools

import jax
import jax.numpy as jnp
import numpy as np
from jax import lax
from jax.experimental import pallas as pl
from jax.experimental.pallas import tpu as pltpu
from jax.experimental.pallas import tpu_sc as plsc


def _lr_sparsecore(idx_arr, flat_table, batch):
    info = plsc.get_sparse_core_info()
    nc, ns, nl = info.num_cores, info.num_subcores, info.num_lanes
    nw = nc * ns
    n_per_w = idx_arr.shape[1]
    bpw = batch // nw
    nfields = n_per_w // bpw

    mesh = plsc.VectorSubcoreMesh(core_axis_name="c", subcore_axis_name="s")

    @functools.partial(
        pl.kernel,
        mesh=mesh,
        out_type=jax.ShapeDtypeStruct((batch,), jnp.float32),
        scratch_types=[
            pltpu.VMEM((n_per_w,), jnp.int32),
            pltpu.VMEM((n_per_w,), jnp.float32),
            pltpu.VMEM((bpw,), jnp.float32),
            pltpu.SemaphoreType.DMA,
        ],
    )
    def lr_kernel(idx_hbm, table_hbm, out_hbm, idx_v, w_v, acc_v, sem):
        wid = lax.axis_index("s") * nc + lax.axis_index("c")
        pltpu.sync_copy(idx_hbm.at[wid], idx_v)
        pltpu.async_copy(table_hbm.at[idx_v], w_v, sem).wait()

        def body(k, _):
            base = k * nl
            acc = w_v[pl.ds(base, nl)]
            for f in range(1, nfields):
                acc = acc + w_v[pl.ds(f * bpw + base, nl)]
            acc_v[pl.ds(base, nl)] = acc
            return 0

        lax.fori_loop(0, bpw // nl, body, 0)
        pltpu.sync_copy(acc_v, out_hbm.at[pl.ds(wid * bpw, bpw)])

    return lr_kernel(idx_arr, flat_table)


def _interaction_tc(emb2d, sel):
    batch, fd = emb2d.shape
    d = sel.shape[1]
    blk = 1024
    rpb = blk // 128

    def body(emb_ref, sel_ref, out_ref):
        x = emb_ref[...]
        sum_sq = jnp.sum(x * x, axis=1)
        s = jnp.dot(x, sel_ref[...],
                    preferred_element_type=jnp.float32)
        dots = 0.5 * (jnp.sum(s * s, axis=1) - sum_sq)
        out_ref[...] = dots.reshape(rpb, 128)

    return pl.pallas_call(
        body,
        grid=(batch // blk,),
        in_specs=[
            pl.BlockSpec((blk, fd), lambda i: (i, 0)),
            pl.BlockSpec((fd, d), lambda i: (0, 0)),
        ],
        out_specs=pl.BlockSpec((rpb, 128), lambda i: (i, 0)),
        out_shape=jax.ShapeDtypeStruct((batch // 128, 128), jnp.float32),
    )(emb2d, sel)


def _combine_tc(dots128, lr128, bias11):
    rows = dots128.shape[0]

    def body(d_ref, l_ref, bias_ref, out_ref):
        out_ref[...] = d_ref[...] + l_ref[...] + bias_ref[0, 0]

    return pl.pallas_call(
        body,
        grid=(1,),
        in_specs=[
            pl.BlockSpec((rows, 128), lambda i: (0, 0)),
            pl.BlockSpec((rows, 128), lambda i: (0, 0)),
            pl.BlockSpec((1, 1), lambda i: (0, 0)),
        ],
        out_specs=pl.BlockSpec((rows, 128), lambda i: (0, 0)),
        out_shape=jax.ShapeDtypeStruct((rows, 128), jnp.float32),
    )(dots128, lr128, bias11)


def kernel(X, feature_emb, lr_table, bias):
    batch, nfields = X.shape
    vocab = lr_table.shape[1]
    d = feature_emb.shape[2]
    fd = nfields * d

    info = plsc.get_sparse_core_info()
    nw = info.num_cores * info.num_subcores
    bpw = batch // nw

    idx = X + jnp.arange(nfields, dtype=X.dtype)[None, :] * vocab
    idx_arr = (
        idx.reshape(nw, bpw, nfields)
        .transpose(0, 2, 1)
        .reshape(nw, nfields * bpw)
    )

    lr_vec = _lr_sparsecore(idx_arr, lr_table.reshape(-1), batch)

    sel = jnp.asarray(
        np.tile(np.eye(d, dtype=np.float32), (nfields, 1)))
    dots128 = _interaction_tc(feature_emb.reshape(batch, fd), sel)
    out128 = _combine_tc(dots128, lr_vec.reshape(batch // 128, 128),
                         bias.reshape(1, 1))
    return out128.reshape(batch, 1)
